# baseline (speedup 1.0000x reference)
import functools

import jax
import jax.numpy as jnp
from jax import lax
from jax.experimental import pallas as pl
from jax.experimental.pallas import tpu as pltpu
from jax.experimental.pallas import tpu_sc as plsc

NUM_SPEAKERS = 100
EMB = 512
BATCH = 16384

_info = plsc.get_sparse_core_info()
_NC, _NS = _info.num_cores, _info.num_subcores
NW = _NC * _NS
B_PER_W = BATCH // NW
CH = 64
NCHUNK = B_PER_W // CH
NBUF = 2
LANES = 16
VPR = EMB // LANES


@functools.partial(
    pl.kernel,
    mesh=plsc.VectorSubcoreMesh(core_axis_name="c", subcore_axis_name="s"),
    out_type=jax.ShapeDtypeStruct((BATCH, EMB), jnp.float32),
    scratch_types=[
        pltpu.VMEM((B_PER_W,), jnp.int32),
        pltpu.VMEM((NUM_SPEAKERS, EMB), jnp.float32),
        pltpu.VMEM((NBUF, CH, EMB), jnp.float32),
        pltpu.SemaphoreType.DMA,
        pltpu.SemaphoreType.DMA,
    ],
)
def _sc_lookup(idx_hbm, table_hbm, out_hbm, idx_v, table_v, rows_v, sw0, sw1):
    wid = lax.axis_index("s") * _NC + lax.axis_index("c")
    base = wid * B_PER_W
    pltpu.sync_copy(table_hbm, table_v)
    pltpu.sync_copy(idx_hbm.at[pl.ds(base, B_PER_W)], idx_v)

    sw = (sw0, sw1)

    def fill_chunk(j, b):
        def group(g, _):
            iv = idx_v[pl.ds(j * CH + g * LANES, LANES)]
            for k in range(LANES):
                r = iv[k]
                src = table_v.at[r]
                dst = rows_v.at[b, g * LANES + k]
                for c in range(VPR):
                    dst[pl.ds(c * LANES, LANES)] = src[pl.ds(c * LANES, LANES)]
            return 0

        lax.fori_loop(0, CH // LANES, group, 0, unroll=False)

    def drain(b):
        pltpu.make_async_copy(
            rows_v.at[b], out_hbm.at[pl.ds(0, CH)], sw[b]).wait()

    def pair(p, _):
        for b in range(NBUF):
            j = p * NBUF + b

            @pl.when(p > 0)
            def _():
                drain(b)

            fill_chunk(j, b)
            pltpu.async_copy(
                rows_v.at[b], out_hbm.at[pl.ds(base + j * CH, CH)], sw[b])
        return 0

    lax.fori_loop(0, NCHUNK // NBUF, pair, 0, unroll=False)
    for b in range(NBUF):
        drain(b)


def kernel(speaker_id, table):
    return _sc_lookup(speaker_id.astype(jnp.int32), table)


# speedup vs baseline: 1.5245x; 1.5245x over previous
import functools

import jax
import jax.numpy as jnp
from jax import lax
from jax.experimental import pallas as pl
from jax.experimental.pallas import tpu as pltpu
from jax.experimental.pallas import tpu_sc as plsc

NUM_SPEAKERS = 100
EMB = 512
BATCH = 16384

_info = plsc.get_sparse_core_info()
_NC, _NS = _info.num_cores, _info.num_subcores
NW = _NC * _NS
B_PER_W = BATCH // NW
CH = 64
NCHUNK = B_PER_W // CH
NBUF = 2
LANES = 16
VPR = EMB // LANES


@functools.partial(
    pl.kernel,
    mesh=plsc.VectorSubcoreMesh(core_axis_name="c", subcore_axis_name="s"),
    out_type=jax.ShapeDtypeStruct((BATCH, EMB), jnp.float32),
    compiler_params=pltpu.CompilerParams(needs_layout_passes=False),
    scratch_types=[
        pltpu.VMEM((B_PER_W,), jnp.int32),
        pltpu.VMEM((NUM_SPEAKERS, EMB), jnp.float32),
        pltpu.VMEM((NBUF, CH, EMB), jnp.float32),
        pltpu.SemaphoreType.DMA,
        pltpu.SemaphoreType.DMA,
    ],
)
def _sc_lookup(idx_hbm, table_hbm, out_hbm, idx_v, table_v, rows_v, sw0, sw1):
    wid = lax.axis_index("s") * _NC + lax.axis_index("c")
    base = wid * B_PER_W
    pltpu.sync_copy(table_hbm, table_v)
    pltpu.sync_copy(idx_hbm.at[pl.ds(base, B_PER_W)], idx_v)

    sw = (sw0, sw1)

    def fill_chunk(j, b):
        half = VPR // 2

        def group(g, _):
            for k in range(LANES):
                row = g * LANES + k
                rk = plsc.load_gather(
                    idx_v, [jnp.full((LANES,), j * CH + row, jnp.int32)])
                for h in range(2):
                    vals = []
                    for c in range(half):
                        col = (lax.iota(jnp.int32, LANES)
                               + (h * half + c) * LANES)
                        vals.append(plsc.load_gather(table_v, [rk, col]))
                    for c in range(half):
                        rows_v[b, row, pl.ds((h * half + c) * LANES, LANES)] = (
                            vals[c])
            return 0

        lax.fori_loop(0, CH // LANES, group, 0, unroll=False)

    def drain(b):
        pltpu.make_async_copy(
            rows_v.at[b], out_hbm.at[pl.ds(0, CH)], sw[b]).wait()

    def pair(p, _):
        for b in range(NBUF):
            j = p * NBUF + b

            @pl.when(p > 0)
            def _():
                drain(b)

            fill_chunk(j, b)
            pltpu.async_copy(
                rows_v.at[b], out_hbm.at[pl.ds(base + j * CH, CH)], sw[b])
        return 0

    lax.fori_loop(0, NCHUNK // NBUF, pair, 0, unroll=False)
    for b in range(NBUF):
        drain(b)


def kernel(speaker_id, table):
    return _sc_lookup(speaker_id.astype(jnp.int32), table)


# speedup vs baseline: 1.7877x; 1.1727x over previous
import functools

import jax
import jax.numpy as jnp
from jax import lax
from jax.experimental import pallas as pl
from jax.experimental.pallas import tpu as pltpu
from jax.experimental.pallas import tpu_sc as plsc

NUM_SPEAKERS = 100
EMB = 512
BATCH = 16384

_info = plsc.get_sparse_core_info()
_NC, _NS = _info.num_cores, _info.num_subcores
NW = _NC * _NS
B_PER_W = BATCH // NW
CH = 64
NCHUNK = B_PER_W // CH
NBUF = 2
LANES = 16
VPR = EMB // LANES


@functools.partial(
    pl.kernel,
    mesh=plsc.VectorSubcoreMesh(core_axis_name="c", subcore_axis_name="s"),
    out_type=jax.ShapeDtypeStruct((BATCH, EMB), jnp.float32),
    compiler_params=pltpu.CompilerParams(needs_layout_passes=False),
    scratch_types=[
        pltpu.VMEM((B_PER_W,), jnp.int32),
        pltpu.VMEM((NUM_SPEAKERS, EMB), jnp.float32),
        pltpu.VMEM((NBUF, CH, EMB), jnp.float32),
        pltpu.SemaphoreType.DMA,
        pltpu.SemaphoreType.DMA,
    ],
)
def _sc_lookup(idx_hbm, table_hbm, out_hbm, idx_v, table_v, rows_v, sw0, sw1):
    wid = lax.axis_index("s") * _NC + lax.axis_index("c")
    base = wid * B_PER_W
    pltpu.sync_copy(table_hbm, table_v)
    pltpu.sync_copy(idx_hbm.at[pl.ds(base, B_PER_W)], idx_v)

    sw = (sw0, sw1)

    def fill_chunk(j, b):
        half = VPR // 2

        def group(g, _):
            iv = idx_v[pl.ds(j * CH + g * LANES, LANES)]
            for k in range(LANES):
                row = g * LANES + k
                rk = jnp.take_along_axis(
                    iv, jnp.full((LANES,), k, jnp.int32), axis=0)

                def load(c):
                    col = lax.iota(jnp.int32, LANES) + c * LANES
                    return plsc.load_gather(table_v, [rk, col])

                vals = [load(c) for c in range(half)]
                for c in range(VPR):
                    if c + half < VPR:
                        vals.append(load(c + half))
                    rows_v[b, row, pl.ds(c * LANES, LANES)] = vals[c]
            return 0

        lax.fori_loop(0, CH // LANES, group, 0, unroll=False)

    def drain(b):
        pltpu.make_async_copy(
            rows_v.at[b], out_hbm.at[pl.ds(0, CH)], sw[b]).wait()

    def pair(p, _):
        for b in range(NBUF):
            j = p * NBUF + b

            @pl.when(p > 0)
            def _():
                drain(b)

            fill_chunk(j, b)
            pltpu.async_copy(
                rows_v.at[b], out_hbm.at[pl.ds(base + j * CH, CH)], sw[b])
        return 0

    lax.fori_loop(0, NCHUNK // NBUF, pair, 0, unroll=False)
    for b in range(NBUF):
        drain(b)


def kernel(speaker_id, table):
    return _sc_lookup(speaker_id.astype(jnp.int32), table)


# speedup vs baseline: 1.9431x; 1.0870x over previous
import functools

import jax
import jax.numpy as jnp
from jax import lax
from jax.experimental import pallas as pl
from jax.experimental.pallas import tpu as pltpu
from jax.experimental.pallas import tpu_sc as plsc

NUM_SPEAKERS = 100
EMB = 512
BATCH = 16384

_info = plsc.get_sparse_core_info()
_NC, _NS = _info.num_cores, _info.num_subcores
NW = _NC * _NS
B_PER_W = BATCH // NW
CH = 64
NCHUNK = B_PER_W // CH
NBUF = 2
LANES = 16
VPR = EMB // LANES


@functools.partial(
    pl.kernel,
    mesh=plsc.VectorSubcoreMesh(core_axis_name="c", subcore_axis_name="s"),
    out_type=jax.ShapeDtypeStruct((BATCH, EMB), jnp.float32),
    compiler_params=pltpu.CompilerParams(needs_layout_passes=False),
    scratch_types=[
        pltpu.VMEM((B_PER_W,), jnp.int32),
        pltpu.VMEM((NUM_SPEAKERS, EMB), jnp.float32),
        pltpu.VMEM((NBUF, CH, EMB), jnp.float32),
        pltpu.SemaphoreType.DMA,
        pltpu.SemaphoreType.DMA,
    ],
)
def _sc_lookup(idx_hbm, table_hbm, out_hbm, idx_v, table_v, rows_v, sw0, sw1):
    wid = lax.axis_index("s") * _NC + lax.axis_index("c")
    base = wid * B_PER_W
    pltpu.sync_copy(table_hbm, table_v)
    pltpu.sync_copy(idx_hbm.at[pl.ds(base, B_PER_W)], idx_v)

    sw = (sw0, sw1)

    def fill_chunk(j, b):
        half = VPR // 2

        LAG = 8

        def group(g, _):
            iv = idx_v[pl.ds(j * CH + g * LANES, LANES)]

            def bcast(k):
                return jnp.take_along_axis(
                    iv, jnp.full((LANES,), k, jnp.int32), axis=0)

            def load(rk, c):
                col = lax.iota(jnp.int32, LANES) + c * LANES
                return plsc.load_gather(table_v, [rk, col])

            rk = {0: bcast(0)}
            pend = []
            for k in range(LANES):
                if k + 1 < LANES:
                    rk[k + 1] = bcast(k + 1)
                for c in range(VPR):
                    pend.append((k, c, load(rk[k], c)))
                    if len(pend) > LAG:
                        kk, cc, vv = pend.pop(0)
                        rows_v[b, g * LANES + kk,
                               pl.ds(cc * LANES, LANES)] = vv
            for kk, cc, vv in pend:
                rows_v[b, g * LANES + kk, pl.ds(cc * LANES, LANES)] = vv
            return 0

        lax.fori_loop(0, CH // LANES, group, 0, unroll=False)

    def drain(b):
        pltpu.make_async_copy(
            rows_v.at[b], out_hbm.at[pl.ds(0, CH)], sw[b]).wait()

    def pair(p, _):
        for b in range(NBUF):
            j = p * NBUF + b

            @pl.when(p > 0)
            def _():
                drain(b)

            fill_chunk(j, b)
            pltpu.async_copy(
                rows_v.at[b], out_hbm.at[pl.ds(base + j * CH, CH)], sw[b])
        return 0

    lax.fori_loop(0, NCHUNK // NBUF, pair, 0, unroll=False)
    for b in range(NBUF):
        drain(b)


def kernel(speaker_id, table):
    return _sc_lookup(speaker_id.astype(jnp.int32), table)


# speedup vs baseline: 2.2711x; 1.1688x over previous
import functools

import jax
import jax.numpy as jnp
from jax import lax
from jax.experimental import pallas as pl
from jax.experimental.pallas import tpu as pltpu
from jax.experimental.pallas import tpu_sc as plsc

NUM_SPEAKERS = 100
EMB = 512
BATCH = 16384

_info = plsc.get_sparse_core_info()
_NC, _NS = _info.num_cores, _info.num_subcores
NW = _NC * _NS
B_PER_W = BATCH // NW
CH = 64
NCHUNK = B_PER_W // CH
NBUF = 2
LANES = 16
VPR = EMB // LANES


@functools.partial(
    pl.kernel,
    mesh=plsc.VectorSubcoreMesh(core_axis_name="c", subcore_axis_name="s"),
    out_type=jax.ShapeDtypeStruct((BATCH, EMB), jnp.float32),
    compiler_params=pltpu.CompilerParams(needs_layout_passes=False),
    scratch_types=[
        pltpu.VMEM((B_PER_W,), jnp.int32),
        pltpu.VMEM((NUM_SPEAKERS * EMB,), jnp.float32),
        pltpu.VMEM((NBUF, CH, EMB), jnp.float32),
        pltpu.SemaphoreType.DMA,
        pltpu.SemaphoreType.DMA,
    ],
)
def _sc_lookup(idx_hbm, table_hbm, out_hbm, idx_v, table_v, rows_v, sw0, sw1):
    wid = lax.axis_index("s") * _NC + lax.axis_index("c")
    base = wid * B_PER_W
    pltpu.sync_copy(table_hbm, table_v)
    pltpu.sync_copy(idx_hbm.at[pl.ds(base, B_PER_W)], idx_v)

    sw = (sw0, sw1)

    def fill_chunk(j, b):
        half = VPR // 2

        LAG = 8

        def group(g, _):
            iv = idx_v[pl.ds(j * CH + g * LANES, LANES)]

            def bcast(k):
                rk = jnp.take_along_axis(
                    iv, jnp.full((LANES,), k, jnp.int32), axis=0)
                return rk * EMB + lax.iota(jnp.int32, LANES)

            def load(rb, c):
                view = table_v.at[pl.ds(c * LANES, NUM_SPEAKERS * EMB - c * LANES)]
                return plsc.load_gather(view, [rb])

            rk = {0: bcast(0)}
            pend = []
            for k in range(LANES):
                if k + 1 < LANES:
                    rk[k + 1] = bcast(k + 1)
                for c in range(VPR):
                    pend.append((k, c, load(rk[k], c)))
                    if len(pend) > LAG:
                        kk, cc, vv = pend.pop(0)
                        rows_v[b, g * LANES + kk,
                               pl.ds(cc * LANES, LANES)] = vv
            for kk, cc, vv in pend:
                rows_v[b, g * LANES + kk, pl.ds(cc * LANES, LANES)] = vv
            return 0

        lax.fori_loop(0, CH // LANES, group, 0, unroll=False)

    def drain(b):
        pltpu.make_async_copy(
            rows_v.at[b], out_hbm.at[pl.ds(0, CH)], sw[b]).wait()

    def pair(p, _):
        for b in range(NBUF):
            j = p * NBUF + b

            @pl.when(p > 0)
            def _():
                drain(b)

            fill_chunk(j, b)
            pltpu.async_copy(
                rows_v.at[b], out_hbm.at[pl.ds(base + j * CH, CH)], sw[b])
        return 0

    lax.fori_loop(0, NCHUNK // NBUF, pair, 0, unroll=False)
    for b in range(NBUF):
        drain(b)


def kernel(speaker_id, table):
    return _sc_lookup(speaker_id.astype(jnp.int32), table.reshape(-1))
